# initial kernel scaffold (unmeasured)
import jax
import jax.numpy as jnp
from jax import lax
from jax.experimental import pallas as pl
from jax.experimental.pallas import tpu as pltpu

NC = 16
LAG = 2


def kernel(x, dy):
    k, d = x.shape
    _, f = dy.shape
    d_half = d // 2
    f_half = f // 2
    cw = f_half // NC
    n_xc = d // cw

    def body(x_hbm, dy_hbm, out_ref, xbf, stage, psend, recvx,
             stage_sems, sendx_sems, recvx_sems, sendy_sems, recvy_sems):
        px = lax.axis_index("x")
        py = lax.axis_index("y")

        x_dmas = [
            pltpu.make_async_copy(
                x_hbm.at[:, pl.ds(i * cw, cw)],
                stage.at[i % 2],
                stage_sems.at[i % 2],
            )
            for i in range(n_xc)
        ]
        dy_dmas = [
            pltpu.make_async_copy(
                dy_hbm.at[:, pl.ds(py * f_half + c * cw, cw)],
                stage.at[(n_xc + c) % 2],
                stage_sems.at[(n_xc + c) % 2],
            )
            for c in range(NC)
        ]
        seq = x_dmas + dy_dmas

        rs_rdmas = [
            pltpu.make_async_remote_copy(
                src_ref=psend.at[:, pl.ds(c * cw, cw)],
                dst_ref=recvx.at[:, pl.ds(c * cw, cw)],
                send_sem=sendx_sems.at[c],
                recv_sem=recvx_sems.at[c],
                device_id=(1 - px, py),
                device_id_type=pl.DeviceIdType.MESH,
            )
            for c in range(NC)
        ]
        ag_rdmas = [
            pltpu.make_async_remote_copy(
                src_ref=out_ref.at[:, pl.ds(py * f_half + c * cw, cw)],
                dst_ref=out_ref.at[:, pl.ds(py * f_half + c * cw, cw)],
                send_sem=sendy_sems.at[c],
                recv_sem=recvy_sems.at[c],
                device_id=(px, 1 - py),
                device_id_type=pl.DeviceIdType.MESH,
            )
            for c in range(NC)
        ]

        seq[0].start()
        for i in range(n_xc):
            seq[i + 1].start()
            x_dmas[i].wait()
            xbf[:, pl.ds(i * cw, cw)] = stage[i % 2].astype(jnp.bfloat16)

        barrier = pltpu.get_barrier_semaphore()
        for peer in ((1 - px, py), (px, 1 - py)):
            pl.semaphore_signal(
                barrier, inc=1, device_id=peer,
                device_id_type=pl.DeviceIdType.MESH,
            )
        pl.semaphore_wait(barrier, 2)

        def consume_rs(j):
            rs_rdmas[j].wait_recv()
            col = pl.ds(py * f_half + j * cw, cw)
            out_ref[:, col] = out_ref[:, col] + recvx[:, pl.ds(j * cw, cw)]
            ag_rdmas[j].start()

        for c in range(NC):
            if c + 1 < NC:
                dy_dmas[c + 1].start()
            dy_dmas[c].wait()
            db = stage[(n_xc + c) % 2].astype(jnp.bfloat16)
            chunk = pl.ds(c * cw, cw)
            p_peer = lax.dot_general(
                xbf[:, pl.ds((1 - px) * d_half, d_half)], db,
                (((0,), (0,)), ((), ())),
                preferred_element_type=jnp.float32,
            )
            psend[:, chunk] = p_peer.astype(jnp.bfloat16)
            rs_rdmas[c].start()
            p_mine = lax.dot_general(
                xbf[:, pl.ds(px * d_half, d_half)], db,
                (((0,), (0,)), ((), ())),
                preferred_element_type=jnp.float32,
            )
            out_ref[:, pl.ds(py * f_half + c * cw, cw)] = p_mine.astype(
                jnp.bfloat16
            )
            if c >= LAG:
                consume_rs(c - LAG)

        for j in range(NC - LAG, NC):
            consume_rs(j)

        for c in range(NC):
            ag_rdmas[c].wait_recv()
        for c in range(NC):
            rs_rdmas[c].wait_send()
            ag_rdmas[c].wait_send()

    return pl.pallas_call(
        body,
        out_shape=jax.ShapeDtypeStruct((d_half, f), jnp.bfloat16),
        in_specs=[
            pl.BlockSpec(memory_space=pltpu.ANY),
            pl.BlockSpec(memory_space=pltpu.ANY),
        ],
        out_specs=pl.BlockSpec(memory_space=pltpu.VMEM),
        scratch_shapes=[
            pltpu.VMEM((k, d), jnp.bfloat16),
            pltpu.VMEM((2, k, cw), jnp.float32),
            pltpu.VMEM((d_half, f_half), jnp.bfloat16),
            pltpu.VMEM((d_half, f_half), jnp.bfloat16),
            pltpu.SemaphoreType.DMA((2,)),
            pltpu.SemaphoreType.DMA((NC,)),
            pltpu.SemaphoreType.DMA((NC,)),
            pltpu.SemaphoreType.DMA((NC,)),
            pltpu.SemaphoreType.DMA((NC,)),
        ],
        compiler_params=pltpu.CompilerParams(collective_id=0),
    )(x, dy)


# baseline (device time: 124193 ns/iter reference)
import jax
import jax.numpy as jnp
from jax import lax
from jax.experimental import pallas as pl
from jax.experimental.pallas import tpu as pltpu

NC = 16
LAG = 2


def kernel(x, dy):
    k, d = x.shape
    _, f = dy.shape
    d_half = d // 2
    f_half = f // 2
    cw = f_half // NC
    n_xc = d // cw

    def body(x_hbm, dy_hbm, out_ref, xbf, stage, psend, recvx,
             stage_sems, sendx_sems, recvx_sems, sendy_sems, recvy_sems):
        px = lax.axis_index("x")
        py = lax.axis_index("y")

        x_dmas = [
            pltpu.make_async_copy(
                x_hbm.at[:, pl.ds(i * cw, cw)],
                stage.at[i % 2],
                stage_sems.at[i % 2],
            )
            for i in range(n_xc)
        ]
        dy_dmas = [
            pltpu.make_async_copy(
                dy_hbm.at[:, pl.ds(py * f_half + c * cw, cw)],
                stage.at[(n_xc + c) % 2],
                stage_sems.at[(n_xc + c) % 2],
            )
            for c in range(NC)
        ]
        seq = x_dmas + dy_dmas

        rs_rdmas = [
            pltpu.make_async_remote_copy(
                src_ref=psend.at[:, pl.ds(c * cw, cw)],
                dst_ref=recvx.at[:, pl.ds(c * cw, cw)],
                send_sem=sendx_sems.at[c],
                recv_sem=recvx_sems.at[c],
                device_id=(1 - px, py),
                device_id_type=pl.DeviceIdType.MESH,
            )
            for c in range(NC)
        ]
        ag_rdmas = [
            pltpu.make_async_remote_copy(
                src_ref=out_ref.at[:, pl.ds(py * f_half + c * cw, cw)],
                dst_ref=out_ref.at[:, pl.ds(py * f_half + c * cw, cw)],
                send_sem=sendy_sems.at[c],
                recv_sem=recvy_sems.at[c],
                device_id=(px, 1 - py),
                device_id_type=pl.DeviceIdType.MESH,
            )
            for c in range(NC)
        ]

        seq[0].start()
        for i in range(n_xc):
            seq[i + 1].start()
            x_dmas[i].wait()
            xbf[:, pl.ds(i * cw, cw)] = stage[i % 2].astype(jnp.bfloat16)

        barrier = pltpu.get_barrier_semaphore()
        for peer in ((1 - px, py), (px, 1 - py)):
            pl.semaphore_signal(
                barrier, inc=1, device_id=peer,
                device_id_type=pl.DeviceIdType.MESH,
            )
        pl.semaphore_wait(barrier, 2)

        def consume_rs(j):
            rs_rdmas[j].wait_recv()
            col = pl.ds(py * f_half + j * cw, cw)
            out_ref[:, col] = out_ref[:, col] + recvx[:, pl.ds(j * cw, cw)]
            ag_rdmas[j].start()

        for c in range(NC):
            if c + 1 < NC:
                dy_dmas[c + 1].start()
            dy_dmas[c].wait()
            db = stage[(n_xc + c) % 2].astype(jnp.bfloat16)
            chunk = pl.ds(c * cw, cw)
            p_peer = lax.dot_general(
                xbf[:, pl.ds((1 - px) * d_half, d_half)], db,
                (((0,), (0,)), ((), ())),
                preferred_element_type=jnp.float32,
            )
            psend[:, chunk] = p_peer.astype(jnp.bfloat16)
            rs_rdmas[c].start()
            p_mine = lax.dot_general(
                xbf[:, pl.ds(px * d_half, d_half)], db,
                (((0,), (0,)), ((), ())),
                preferred_element_type=jnp.float32,
            )
            out_ref[:, pl.ds(py * f_half + c * cw, cw)] = p_mine.astype(
                jnp.bfloat16
            )
            if c >= LAG:
                consume_rs(c - LAG)

        for j in range(NC - LAG, NC):
            consume_rs(j)

        for c in range(NC):
            ag_rdmas[c].wait_recv()
        for c in range(NC):
            rs_rdmas[c].wait_send()
            ag_rdmas[c].wait_send()

    return pl.pallas_call(
        body,
        out_shape=jax.ShapeDtypeStruct((d_half, f), jnp.bfloat16),
        in_specs=[
            pl.BlockSpec(memory_space=pl.ANY),
            pl.BlockSpec(memory_space=pl.ANY),
        ],
        out_specs=pl.BlockSpec(memory_space=pltpu.VMEM),
        scratch_shapes=[
            pltpu.VMEM((k, d), jnp.bfloat16),
            pltpu.VMEM((2, k, cw), jnp.float32),
            pltpu.VMEM((d_half, f_half), jnp.bfloat16),
            pltpu.VMEM((d_half, f_half), jnp.bfloat16),
            pltpu.SemaphoreType.DMA((2,)),
            pltpu.SemaphoreType.DMA((NC,)),
            pltpu.SemaphoreType.DMA((NC,)),
            pltpu.SemaphoreType.DMA((NC,)),
            pltpu.SemaphoreType.DMA((NC,)),
        ],
        compiler_params=pltpu.CompilerParams(collective_id=0),
    )(x, dy)
